# baseline (speedup 1.0000x reference)
import functools

import jax
import jax.numpy as jnp
import numpy as np
from jax import lax
from jax.experimental import pallas as pl
from jax.experimental.pallas import tpu as pltpu
from jax.experimental.pallas import tpu_sc as plsc

VOCAB = 30522
D = 64
NH = 4
DH = D // NH
NL = 2
NE = 32
NC = 4
B = 32
L = 128

_SC_CORES = 2
_SC_SUBCORES = 16
_NW = _SC_CORES * _SC_SUBCORES
_ROWS_PER_W = (B * L) // _NW


def _sinusoidal(length, dim):
    pos = np.arange(length)[:, None].astype(np.float32)
    i = np.arange(dim)[None, :].astype(np.float32)
    angle = pos / np.power(10000.0, (2.0 * np.floor(i / 2.0)) / dim)
    pe = np.zeros((length, dim), dtype=np.float32)
    pe[:, 0::2] = np.sin(angle[:, 0::2])
    pe[:, 1::2] = np.cos(angle[:, 1::2])
    return pe


_PE = _sinusoidal(L, D)


@functools.lru_cache(maxsize=1)
def _make_word_gather():
    mesh = plsc.VectorSubcoreMesh(core_axis_name="c", subcore_axis_name="s")

    @functools.partial(
        pl.kernel,
        mesh=mesh,
        out_type=jax.ShapeDtypeStruct((B * L, 128), jnp.float32),
        scratch_types=[
            pltpu.VMEM((_ROWS_PER_W,), jnp.int32),
            pltpu.VMEM((_ROWS_PER_W, 128), jnp.float32),
            pltpu.SemaphoreType.DMA,
        ],
    )
    def gather_kernel(table_hbm, idx_hbm, out_hbm, idx_v, rows_v, sem):
        wid = lax.axis_index("s") * _SC_CORES + lax.axis_index("c")
        base = wid * _ROWS_PER_W
        pltpu.sync_copy(idx_hbm.at[pl.ds(base, _ROWS_PER_W)], idx_v)
        pltpu.async_copy(table_hbm.at[idx_v], rows_v, sem).wait()
        pltpu.sync_copy(rows_v, out_hbm.at[pl.ds(base, _ROWS_PER_W)])

    return gather_kernel


def _word_gather(table, ids):
    return _make_word_gather()(table, ids)


_G = 2


def _tc_body(h0_ref, adjt_ref, ett_ref, pe_ref, eemb_ref, wq_ref,
             wk_ref, wv_ref, wo_ref, lng_ref, lnb_ref, wout_ref, bout_ref,
             out_ref):
    for g in range(_G):
        _graph_forward(g, h0_ref, adjt_ref, ett_ref, pe_ref, eemb_ref,
                       wq_ref, wk_ref, wv_ref, wo_ref, lng_ref, lnb_ref,
                       wout_ref, bout_ref, out_ref)


def _graph_forward(g, h0_ref, adjt_ref, ett_ref, pe_ref, eemb_ref, wq_ref,
                   wk_ref, wv_ref, wo_ref, lng_ref, lnb_ref, wout_ref,
                   bout_ref, out_ref):
    h = h0_ref[g][:, :D] + pe_ref[...]
    adjt = adjt_ref[g]
    ett = ett_ref[g]
    edge_ok = adjt > 0
    scale = jnp.float32(1.0 / np.sqrt(DH))

    for i in range(NL):
        Wq = wq_ref[i]
        Wk = wk_ref[i]
        Wv = wv_ref[i]
        q = jnp.dot(h, Wq, preferred_element_type=jnp.float32)
        k = jnp.dot(h, Wk, preferred_element_type=jnp.float32)
        v = jnp.dot(h, Wv, preferred_element_type=jnp.float32)
        eproj = jnp.dot(eemb_ref[...], Wk,
                        preferred_element_type=jnp.float32)
        sts = []
        qets = []
        for hh in range(NH):
            sl = slice(hh * DH, (hh + 1) * DH)
            qets.append(lax.dot_general(eproj[:, sl], q[:, sl],
                                        (((1,), (1,)), ((), ())),
                                        preferred_element_type=jnp.float32))
            sts.append(lax.dot_general(k[:, sl], q[:, sl],
                                       (((1,), (1,)), ((), ())),
                                       preferred_element_type=jnp.float32))
        r = ett.astype(jnp.bfloat16)
        bits = [None] * 5
        for j in (4, 3, 2, 1, 0):
            bits[j] = r >= jnp.bfloat16(1 << j)
            if j:
                r = r - jnp.where(bits[j], jnp.bfloat16(1 << j),
                                  jnp.bfloat16(0))
        for hh in range(NH):
            qet16 = qets[hh].astype(jnp.bfloat16)
            cur = [qet16[t:t + 1, :] for t in range(NE)]
            for j in range(5):
                cur = [jnp.where(bits[j], cur[2 * s + 1], cur[2 * s])
                       for s in range(len(cur) // 2)]
            sts[hh] = sts[hh] + cur[0].astype(jnp.float32)
        outs = []
        ones_row = jnp.ones((1, L), jnp.float32)
        for hh in range(NH):
            st = jnp.where(edge_ok, sts[hh] * scale, jnp.float32(-1e9))
            p = jnp.exp(st)
            denom = jnp.dot(ones_row, p,
                            preferred_element_type=jnp.float32)
            p = p * (1.0 / denom)
            outs.append(lax.dot_general(
                p, v[:, hh * DH:(hh + 1) * DH],
                (((0,), (0,)), ((), ())),
                preferred_element_type=jnp.float32))
        attn_out = jnp.concatenate(outs, axis=1)
        h = h + jnp.dot(attn_out, wo_ref[i],
                        preferred_element_type=jnp.float32)
        m = jnp.mean(h, axis=1, keepdims=True)
        hc = h - m
        var = jnp.mean(hc * hc, axis=1, keepdims=True)
        h = hc * lax.rsqrt(var + 1e-5) * lng_ref[i] + lnb_ref[i]

    cls = h[0:1, :]
    logits = jnp.dot(cls, wout_ref[...],
                     preferred_element_type=jnp.float32) + bout_ref[...]
    out_ref[g] = jnp.pad(logits, ((0, 7), (0, 128 - NC)))


def _tc_forward(h0, adj, et, pe, edge_emb, Wq, Wk, Wv, Wo, ln_g, ln_b,
                W_out, b_out):
    full = lambda shape: pl.BlockSpec(shape, lambda b: (0,) * len(shape))
    padded = pl.pallas_call(
        _tc_body,
        grid=(B // _G,),
        in_specs=[
            pl.BlockSpec((_G, L, 128), lambda b: (b, 0, 0)),
            pl.BlockSpec((_G, L, L), lambda b: (b, 0, 0)),
            pl.BlockSpec((_G, L, L), lambda b: (b, 0, 0)),
            full((L, D)),
            full((NE, D)),
            full((NL, D, D)),
            full((NL, D, D)),
            full((NL, D, D)),
            full((NL, D, D)),
            full((NL, D)),
            full((NL, D)),
            full((D, NC)),
            full((1, NC)),
        ],
        out_specs=pl.BlockSpec((_G, 8, 128), lambda b: (b, 0, 0)),
        out_shape=jax.ShapeDtypeStruct((B, 8, 128), jnp.float32),
    )(h0, adj, et, pe, edge_emb, Wq, Wk, Wv, Wo, ln_g, ln_b, W_out, b_out)
    return padded[:, 0, :NC]


def kernel(word_ids, adj, edge_types, word_emb, edge_emb, Wq, Wk, Wv, Wo,
           ln_g, ln_b, W_out, b_out):
    ids = word_ids.reshape(B * L).astype(jnp.int32)
    table = jnp.pad(word_emb, ((0, 0), (0, 128 - D)))
    h0 = _word_gather(table, ids).reshape(B, L, 128)
    pe = jnp.asarray(_PE)
    adjt = jnp.swapaxes(adj, 1, 2)
    ett = jnp.swapaxes(edge_types, 1, 2)
    return _tc_forward(h0, adjt, ett, pe, edge_emb, Wq, Wk, Wv, Wo,
                       ln_g, ln_b, W_out, b_out.reshape(1, NC))


# speedup vs baseline: 1.0507x; 1.0507x over previous
import functools

import jax
import jax.numpy as jnp
import numpy as np
from jax import lax
from jax.experimental import pallas as pl
from jax.experimental.pallas import tpu as pltpu
from jax.experimental.pallas import tpu_sc as plsc

VOCAB = 30522
D = 64
NH = 4
DH = D // NH
NL = 2
NE = 32
NC = 4
B = 32
L = 128

_SC_CORES = 2
_SC_SUBCORES = 16
_NW = _SC_CORES * _SC_SUBCORES
_ROWS_PER_W = (B * L) // _NW


def _sinusoidal(length, dim):
    pos = np.arange(length)[:, None].astype(np.float32)
    i = np.arange(dim)[None, :].astype(np.float32)
    angle = pos / np.power(10000.0, (2.0 * np.floor(i / 2.0)) / dim)
    pe = np.zeros((length, dim), dtype=np.float32)
    pe[:, 0::2] = np.sin(angle[:, 0::2])
    pe[:, 1::2] = np.cos(angle[:, 1::2])
    return pe


_PE = _sinusoidal(L, D)


@functools.lru_cache(maxsize=1)
def _make_word_gather():
    mesh = plsc.VectorSubcoreMesh(core_axis_name="c", subcore_axis_name="s")

    @functools.partial(
        pl.kernel,
        mesh=mesh,
        out_type=jax.ShapeDtypeStruct((B * L, 128), jnp.float32),
        scratch_types=[
            pltpu.VMEM((_ROWS_PER_W,), jnp.int32),
            pltpu.VMEM((_ROWS_PER_W, 128), jnp.float32),
            pltpu.SemaphoreType.DMA,
        ],
    )
    def gather_kernel(table_hbm, idx_hbm, out_hbm, idx_v, rows_v, sem):
        wid = lax.axis_index("s") * _SC_CORES + lax.axis_index("c")
        base = wid * _ROWS_PER_W
        pltpu.sync_copy(idx_hbm.at[pl.ds(base, _ROWS_PER_W)], idx_v)
        pltpu.async_copy(table_hbm.at[idx_v], rows_v, sem).wait()
        pltpu.sync_copy(rows_v, out_hbm.at[pl.ds(base, _ROWS_PER_W)])

    return gather_kernel


def _word_gather(table, ids):
    return _make_word_gather()(table, ids)


_G = 2


def _tc_body(h0_ref, adjt_ref, ett_ref, pe_ref, eemb_ref, wq_ref,
             wk_ref, wv_ref, wo_ref, lng_ref, lnb_ref, wout_ref, bout_ref,
             out_ref):
    for g in range(_G):
        _graph_forward(g, h0_ref, adjt_ref, ett_ref, pe_ref, eemb_ref,
                       wq_ref, wk_ref, wv_ref, wo_ref, lng_ref, lnb_ref,
                       wout_ref, bout_ref, out_ref)


def _graph_forward(g, h0_ref, adjt_ref, ett_ref, pe_ref, eemb_ref, wq_ref,
                   wk_ref, wv_ref, wo_ref, lng_ref, lnb_ref, wout_ref,
                   bout_ref, out_ref):
    h = h0_ref[g][:, :D] + pe_ref[...]
    adjt = adjt_ref[g]
    ett = ett_ref[g]
    edge_ok = adjt > 0
    scale = jnp.float32(1.0 / np.sqrt(DH))

    for i in range(NL):
        Wq = wq_ref[i]
        Wk = wk_ref[i]
        Wv = wv_ref[i]
        q = jnp.dot(h, Wq, preferred_element_type=jnp.float32)
        k = jnp.dot(h, Wk, preferred_element_type=jnp.float32)
        v = jnp.dot(h, Wv, preferred_element_type=jnp.float32)
        eproj = jnp.dot(eemb_ref[...], Wk,
                        preferred_element_type=jnp.float32)
        sts = []
        qets = []
        for hh in range(NH):
            sl = slice(hh * DH, (hh + 1) * DH)
            qets.append(lax.dot_general(eproj[:, sl], q[:, sl],
                                        (((1,), (1,)), ((), ())),
                                        preferred_element_type=jnp.float32))
            sts.append(lax.dot_general(k[:, sl], q[:, sl],
                                       (((1,), (1,)), ((), ())),
                                       preferred_element_type=jnp.float32))
        r = ett.astype(jnp.bfloat16)
        bits = [None] * 5
        for j in (4, 3, 2, 1, 0):
            bits[j] = r >= jnp.bfloat16(1 << j)
            if j:
                r = r - jnp.where(bits[j], jnp.bfloat16(1 << j),
                                  jnp.bfloat16(0))
        for hh in range(NH):
            qet16 = qets[hh].astype(jnp.bfloat16)
            cur = [qet16[t:t + 1, :] for t in range(NE)]
            for j in range(5):
                cur = [jnp.where(bits[j], cur[2 * s + 1], cur[2 * s])
                       for s in range(len(cur) // 2)]
            sts[hh] = sts[hh] + cur[0].astype(jnp.float32)
        outs = []
        ones_row = jnp.ones((1, L), jnp.float32)
        for hh in range(NH):
            st = jnp.where(edge_ok, sts[hh] * scale, jnp.float32(-1e9))
            p = jnp.exp(st)
            p = p * (1.0 / jnp.sum(p, axis=0, keepdims=True))
            outs.append(lax.dot_general(
                p, v[:, hh * DH:(hh + 1) * DH],
                (((0,), (0,)), ((), ())),
                preferred_element_type=jnp.float32))
        attn_out = jnp.concatenate(outs, axis=1)
        h = h + jnp.dot(attn_out, wo_ref[i],
                        preferred_element_type=jnp.float32)
        m = jnp.mean(h, axis=1, keepdims=True)
        hc = h - m
        var = jnp.mean(hc * hc, axis=1, keepdims=True)
        h = hc * lax.rsqrt(var + 1e-5) * lng_ref[i] + lnb_ref[i]

    cls = h[0:1, :]
    logits = jnp.dot(cls, wout_ref[...],
                     preferred_element_type=jnp.float32) + bout_ref[...]
    out_ref[g] = jnp.pad(logits, ((0, 7), (0, 128 - NC)))


def _tc_forward(h0, adj, et, pe, edge_emb, Wq, Wk, Wv, Wo, ln_g, ln_b,
                W_out, b_out):
    full = lambda shape: pl.BlockSpec(shape, lambda b: (0,) * len(shape))
    padded = pl.pallas_call(
        _tc_body,
        grid=(B // _G,),
        in_specs=[
            pl.BlockSpec((_G, L, 128), lambda b: (b, 0, 0)),
            pl.BlockSpec((_G, L, L), lambda b: (b, 0, 0)),
            pl.BlockSpec((_G, L, L), lambda b: (b, 0, 0)),
            full((L, D)),
            full((NE, D)),
            full((NL, D, D)),
            full((NL, D, D)),
            full((NL, D, D)),
            full((NL, D, D)),
            full((NL, D)),
            full((NL, D)),
            full((D, NC)),
            full((1, NC)),
        ],
        out_specs=pl.BlockSpec((_G, 8, 128), lambda b: (b, 0, 0)),
        out_shape=jax.ShapeDtypeStruct((B, 8, 128), jnp.float32),
    )(h0, adj, et, pe, edge_emb, Wq, Wk, Wv, Wo, ln_g, ln_b, W_out, b_out)
    return padded[:, 0, :NC]


def kernel(word_ids, adj, edge_types, word_emb, edge_emb, Wq, Wk, Wv, Wo,
           ln_g, ln_b, W_out, b_out):
    ids = word_ids.reshape(B * L).astype(jnp.int32)
    table = jnp.pad(word_emb, ((0, 0), (0, 128 - D)))
    h0 = _word_gather(table, ids).reshape(B, L, 128)
    pe = jnp.asarray(_PE)
    adjt = jnp.swapaxes(adj, 1, 2)
    ett = jnp.swapaxes(edge_types, 1, 2)
    return _tc_forward(h0, adjt, ett, pe, edge_emb, Wq, Wk, Wv, Wo,
                       ln_g, ln_b, W_out, b_out.reshape(1, NC))


# speedup vs baseline: 1.2921x; 1.2298x over previous
import functools

import jax
import jax.numpy as jnp
import numpy as np
from jax import lax
from jax.experimental import pallas as pl
from jax.experimental.pallas import tpu as pltpu
from jax.experimental.pallas import tpu_sc as plsc

VOCAB = 30522
D = 64
NH = 4
DH = D // NH
NL = 2
NE = 32
NC = 4
B = 32
L = 128

_SC_CORES = 2
_SC_SUBCORES = 16
_NW = _SC_CORES * _SC_SUBCORES
_ROWS_PER_W = (B * L) // _NW


def _sinusoidal(length, dim):
    pos = np.arange(length)[:, None].astype(np.float32)
    i = np.arange(dim)[None, :].astype(np.float32)
    angle = pos / np.power(10000.0, (2.0 * np.floor(i / 2.0)) / dim)
    pe = np.zeros((length, dim), dtype=np.float32)
    pe[:, 0::2] = np.sin(angle[:, 0::2])
    pe[:, 1::2] = np.cos(angle[:, 1::2])
    return pe


_PE = _sinusoidal(L, D)


@functools.lru_cache(maxsize=1)
def _make_word_gather():
    mesh = plsc.VectorSubcoreMesh(core_axis_name="c", subcore_axis_name="s")

    @functools.partial(
        pl.kernel,
        mesh=mesh,
        out_type=jax.ShapeDtypeStruct((B * L, 128), jnp.float32),
        scratch_types=[
            pltpu.VMEM((_ROWS_PER_W,), jnp.int32),
            pltpu.VMEM((_ROWS_PER_W, 128), jnp.float32),
            pltpu.SemaphoreType.DMA,
        ],
    )
    def gather_kernel(table_hbm, idx_hbm, out_hbm, idx_v, rows_v, sem):
        wid = lax.axis_index("s") * _SC_CORES + lax.axis_index("c")
        base = wid * _ROWS_PER_W
        pltpu.sync_copy(idx_hbm.at[pl.ds(base, _ROWS_PER_W)], idx_v)
        pltpu.async_copy(table_hbm.at[idx_v], rows_v, sem).wait()
        pltpu.sync_copy(rows_v, out_hbm.at[pl.ds(base, _ROWS_PER_W)])

    return gather_kernel


def _word_gather(table, ids):
    return _make_word_gather()(table, ids)


_G = 2


def _tc_body(h0_ref, adjt_ref, ett_ref, pe_ref, eemb_ref, wq_ref,
             wk_ref, wv_ref, wo_ref, lng_ref, lnb_ref, wout_ref, bout_ref,
             out_ref):
    G = range(_G)
    scale = jnp.float32(1.0 / np.sqrt(DH))
    hs = [h0_ref[g][:, :D] + pe_ref[...] for g in G]
    edge_ok = [adjt_ref[g] > 0 for g in G]
    bits = [[None] * 5 for g in G]
    for g in G:
        r = ett_ref[g].astype(jnp.bfloat16)
        for j in (4, 3, 2, 1, 0):
            bits[g][j] = r >= jnp.bfloat16(1 << j)
            if j:
                r = r - jnp.where(bits[g][j], jnp.bfloat16(1 << j),
                                  jnp.bfloat16(0))

    for i in range(NL):
        Wq = wq_ref[i]
        Wk = wk_ref[i]
        Wv = wv_ref[i]
        eproj = jnp.dot(eemb_ref[...], Wk,
                        preferred_element_type=jnp.float32)
        qs = [jnp.dot(hs[g], Wq, preferred_element_type=jnp.float32)
              for g in G]
        ks = [jnp.dot(hs[g], Wk, preferred_element_type=jnp.float32)
              for g in G]
        vs = [jnp.dot(hs[g], Wv, preferred_element_type=jnp.float32)
              for g in G]
        sts = [[None] * NH for g in G]
        qets = [[None] * NH for g in G]
        for hh in range(NH):
            sl = slice(hh * DH, (hh + 1) * DH)
            for g in G:
                qets[g][hh] = lax.dot_general(
                    eproj[:, sl], qs[g][:, sl], (((1,), (1,)), ((), ())),
                    preferred_element_type=jnp.float32)
                sts[g][hh] = lax.dot_general(
                    ks[g][:, sl], qs[g][:, sl], (((1,), (1,)), ((), ())),
                    preferred_element_type=jnp.float32)
        for hh in range(NH):
            for g in G:
                qet16 = qets[g][hh].astype(jnp.bfloat16)
                cur = [qet16[t:t + 1, :] for t in range(NE)]
                for j in range(5):
                    cur = [jnp.where(bits[g][j], cur[2 * s + 1], cur[2 * s])
                           for s in range(len(cur) // 2)]
                sts[g][hh] = sts[g][hh] + cur[0].astype(jnp.float32)
        outs = [[None] * NH for g in G]
        for hh in range(NH):
            for g in G:
                st = jnp.where(edge_ok[g], sts[g][hh] * scale,
                               jnp.float32(-1e9))
                p = jnp.exp(st)
                p = p * (1.0 / jnp.sum(p, axis=0, keepdims=True))
                outs[g][hh] = lax.dot_general(
                    p, vs[g][:, hh * DH:(hh + 1) * DH],
                    (((0,), (0,)), ((), ())),
                    preferred_element_type=jnp.float32)
        for g in G:
            attn_out = jnp.concatenate(outs[g], axis=1)
            hs[g] = hs[g] + jnp.dot(attn_out, wo_ref[i],
                                    preferred_element_type=jnp.float32)
        for g in G:
            m = jnp.mean(hs[g], axis=1, keepdims=True)
            hc = hs[g] - m
            var = jnp.mean(hc * hc, axis=1, keepdims=True)
            hs[g] = hc * lax.rsqrt(var + 1e-5) * lng_ref[i] + lnb_ref[i]

    for g in G:
        cls = hs[g][0:1, :]
        logits = jnp.dot(cls, wout_ref[...],
                         preferred_element_type=jnp.float32) + bout_ref[...]
        out_ref[g] = jnp.pad(logits, ((0, 7), (0, 128 - NC)))


def _tc_forward(h0, adj, et, pe, edge_emb, Wq, Wk, Wv, Wo, ln_g, ln_b,
                W_out, b_out):
    full = lambda shape: pl.BlockSpec(shape, lambda b: (0,) * len(shape))
    padded = pl.pallas_call(
        _tc_body,
        grid=(B // _G,),
        in_specs=[
            pl.BlockSpec((_G, L, 128), lambda b: (b, 0, 0)),
            pl.BlockSpec((_G, L, L), lambda b: (b, 0, 0)),
            pl.BlockSpec((_G, L, L), lambda b: (b, 0, 0)),
            full((L, D)),
            full((NE, D)),
            full((NL, D, D)),
            full((NL, D, D)),
            full((NL, D, D)),
            full((NL, D, D)),
            full((NL, D)),
            full((NL, D)),
            full((D, NC)),
            full((1, NC)),
        ],
        out_specs=pl.BlockSpec((_G, 8, 128), lambda b: (b, 0, 0)),
        out_shape=jax.ShapeDtypeStruct((B, 8, 128), jnp.float32),
    )(h0, adj, et, pe, edge_emb, Wq, Wk, Wv, Wo, ln_g, ln_b, W_out, b_out)
    return padded[:, 0, :NC]


def kernel(word_ids, adj, edge_types, word_emb, edge_emb, Wq, Wk, Wv, Wo,
           ln_g, ln_b, W_out, b_out):
    ids = word_ids.reshape(B * L).astype(jnp.int32)
    table = jnp.pad(word_emb, ((0, 0), (0, 128 - D)))
    h0 = _word_gather(table, ids).reshape(B, L, 128)
    pe = jnp.asarray(_PE)
    adjt = jnp.swapaxes(adj, 1, 2)
    ett = jnp.swapaxes(edge_types, 1, 2)
    return _tc_forward(h0, adjt, ett, pe, edge_emb, Wq, Wk, Wv, Wo,
                       ln_g, ln_b, W_out, b_out.reshape(1, NC))


# speedup vs baseline: 1.4877x; 1.1513x over previous
import functools

import jax
import jax.numpy as jnp
import numpy as np
from jax import lax
from jax.experimental import pallas as pl
from jax.experimental.pallas import tpu as pltpu
from jax.experimental.pallas import tpu_sc as plsc

VOCAB = 30522
D = 64
NH = 4
DH = D // NH
NL = 2
NE = 32
NC = 4
B = 32
L = 128

_SC_CORES = 2
_SC_SUBCORES = 16
_NW = _SC_CORES * _SC_SUBCORES
_ROWS_PER_W = (B * L) // _NW


def _sinusoidal(length, dim):
    pos = np.arange(length)[:, None].astype(np.float32)
    i = np.arange(dim)[None, :].astype(np.float32)
    angle = pos / np.power(10000.0, (2.0 * np.floor(i / 2.0)) / dim)
    pe = np.zeros((length, dim), dtype=np.float32)
    pe[:, 0::2] = np.sin(angle[:, 0::2])
    pe[:, 1::2] = np.cos(angle[:, 1::2])
    return pe


_PE = _sinusoidal(L, D)


@functools.lru_cache(maxsize=1)
def _make_word_gather():
    mesh = plsc.VectorSubcoreMesh(core_axis_name="c", subcore_axis_name="s")

    @functools.partial(
        pl.kernel,
        mesh=mesh,
        out_type=jax.ShapeDtypeStruct((B * L, 128), jnp.float32),
        scratch_types=[
            pltpu.VMEM((_ROWS_PER_W,), jnp.int32),
            pltpu.VMEM((_ROWS_PER_W, 128), jnp.float32),
            pltpu.SemaphoreType.DMA,
        ],
    )
    def gather_kernel(table_hbm, idx_hbm, out_hbm, idx_v, rows_v, sem):
        wid = lax.axis_index("s") * _SC_CORES + lax.axis_index("c")
        base = wid * _ROWS_PER_W
        pltpu.sync_copy(idx_hbm.at[pl.ds(base, _ROWS_PER_W)], idx_v)
        pltpu.async_copy(table_hbm.at[idx_v], rows_v, sem).wait()
        pltpu.sync_copy(rows_v, out_hbm.at[pl.ds(base, _ROWS_PER_W)])

    return gather_kernel


def _word_gather(table, ids):
    return _make_word_gather()(table, ids)


_G = 4


def _tc_body(h0_ref, adjt_ref, ett_ref, pe_ref, eemb_ref, wq_ref,
             wk_ref, wv_ref, wo_ref, lng_ref, lnb_ref, wout_ref, bout_ref,
             out_ref):
    G = range(_G)
    scale = jnp.float32(1.0 / np.sqrt(DH))
    hs = [h0_ref[g][:, :D] + pe_ref[...] for g in G]
    edge_ok = [adjt_ref[g] > 0 for g in G]
    bits = [[None] * 5 for g in G]
    for g in G:
        r = ett_ref[g].astype(jnp.bfloat16)
        for j in (4, 3, 2, 1, 0):
            bits[g][j] = r >= jnp.bfloat16(1 << j)
            if j:
                r = r - jnp.where(bits[g][j], jnp.bfloat16(1 << j),
                                  jnp.bfloat16(0))

    for i in range(NL):
        Wq = wq_ref[i]
        Wk = wk_ref[i]
        Wv = wv_ref[i]
        eproj = jnp.dot(eemb_ref[...], Wk,
                        preferred_element_type=jnp.float32)
        qs = [jnp.dot(hs[g], Wq, preferred_element_type=jnp.float32)
              for g in G]
        ks = [jnp.dot(hs[g], Wk, preferred_element_type=jnp.float32)
              for g in G]
        vs = [jnp.dot(hs[g], Wv, preferred_element_type=jnp.float32)
              for g in G]
        sts = [[None] * NH for g in G]
        qets = [[None] * NH for g in G]
        for hh in range(NH):
            sl = slice(hh * DH, (hh + 1) * DH)
            for g in G:
                qets[g][hh] = lax.dot_general(
                    eproj[:, sl], qs[g][:, sl], (((1,), (1,)), ((), ())),
                    preferred_element_type=jnp.float32)
                sts[g][hh] = lax.dot_general(
                    ks[g][:, sl], qs[g][:, sl], (((1,), (1,)), ((), ())),
                    preferred_element_type=jnp.float32)
        for hh in range(NH):
            for g in G:
                qet16 = qets[g][hh].astype(jnp.bfloat16)
                cur = [qet16[t:t + 1, :] for t in range(NE)]
                for j in range(5):
                    cur = [jnp.where(bits[g][j], cur[2 * s + 1], cur[2 * s])
                           for s in range(len(cur) // 2)]
                sts[g][hh] = sts[g][hh] + cur[0].astype(jnp.float32)
        outs = [[None] * NH for g in G]
        for hh in range(NH):
            for g in G:
                st = jnp.where(edge_ok[g], sts[g][hh] * scale,
                               jnp.float32(-1e9))
                p = jnp.exp(st)
                p = p * (1.0 / jnp.sum(p, axis=0, keepdims=True))
                outs[g][hh] = lax.dot_general(
                    p, vs[g][:, hh * DH:(hh + 1) * DH],
                    (((0,), (0,)), ((), ())),
                    preferred_element_type=jnp.float32)
        for g in G:
            attn_out = jnp.concatenate(outs[g], axis=1)
            hs[g] = hs[g] + jnp.dot(attn_out, wo_ref[i],
                                    preferred_element_type=jnp.float32)
        for g in G:
            m = jnp.mean(hs[g], axis=1, keepdims=True)
            hc = hs[g] - m
            var = jnp.mean(hc * hc, axis=1, keepdims=True)
            hs[g] = hc * lax.rsqrt(var + 1e-5) * lng_ref[i] + lnb_ref[i]

    for g in G:
        cls = hs[g][0:1, :]
        logits = jnp.dot(cls, wout_ref[...],
                         preferred_element_type=jnp.float32) + bout_ref[...]
        out_ref[g] = jnp.pad(logits, ((0, 7), (0, 128 - NC)))


def _tc_forward(h0, adj, et, pe, edge_emb, Wq, Wk, Wv, Wo, ln_g, ln_b,
                W_out, b_out):
    full = lambda shape: pl.BlockSpec(shape, lambda b: (0,) * len(shape))
    padded = pl.pallas_call(
        _tc_body,
        grid=(B // _G,),
        in_specs=[
            pl.BlockSpec((_G, L, 128), lambda b: (b, 0, 0)),
            pl.BlockSpec((_G, L, L), lambda b: (b, 0, 0)),
            pl.BlockSpec((_G, L, L), lambda b: (b, 0, 0)),
            full((L, D)),
            full((NE, D)),
            full((NL, D, D)),
            full((NL, D, D)),
            full((NL, D, D)),
            full((NL, D, D)),
            full((NL, D)),
            full((NL, D)),
            full((D, NC)),
            full((1, NC)),
        ],
        out_specs=pl.BlockSpec((_G, 8, 128), lambda b: (b, 0, 0)),
        out_shape=jax.ShapeDtypeStruct((B, 8, 128), jnp.float32),
    )(h0, adj, et, pe, edge_emb, Wq, Wk, Wv, Wo, ln_g, ln_b, W_out, b_out)
    return padded[:, 0, :NC]


def kernel(word_ids, adj, edge_types, word_emb, edge_emb, Wq, Wk, Wv, Wo,
           ln_g, ln_b, W_out, b_out):
    ids = word_ids.reshape(B * L).astype(jnp.int32)
    table = jnp.pad(word_emb, ((0, 0), (0, 128 - D)))
    h0 = _word_gather(table, ids).reshape(B, L, 128)
    pe = jnp.asarray(_PE)
    adjt = jnp.swapaxes(adj, 1, 2)
    ett = jnp.swapaxes(edge_types, 1, 2)
    return _tc_forward(h0, adjt, ett, pe, edge_emb, Wq, Wk, Wv, Wo,
                       ln_g, ln_b, W_out, b_out.reshape(1, NC))


# speedup vs baseline: 1.5577x; 1.0470x over previous
import functools

import jax
import jax.numpy as jnp
import numpy as np
from jax import lax
from jax.experimental import pallas as pl
from jax.experimental.pallas import tpu as pltpu
from jax.experimental.pallas import tpu_sc as plsc

VOCAB = 30522
D = 64
NH = 4
DH = D // NH
NL = 2
NE = 32
NC = 4
B = 32
L = 128

_SC_CORES = 2
_SC_SUBCORES = 16
_NW = _SC_CORES * _SC_SUBCORES
_ROWS_PER_W = (B * L) // _NW


def _sinusoidal(length, dim):
    pos = np.arange(length)[:, None].astype(np.float32)
    i = np.arange(dim)[None, :].astype(np.float32)
    angle = pos / np.power(10000.0, (2.0 * np.floor(i / 2.0)) / dim)
    pe = np.zeros((length, dim), dtype=np.float32)
    pe[:, 0::2] = np.sin(angle[:, 0::2])
    pe[:, 1::2] = np.cos(angle[:, 1::2])
    return pe


_PE = _sinusoidal(L, D)


@functools.lru_cache(maxsize=1)
def _make_word_gather():
    mesh = plsc.VectorSubcoreMesh(core_axis_name="c", subcore_axis_name="s")

    @functools.partial(
        pl.kernel,
        mesh=mesh,
        out_type=jax.ShapeDtypeStruct((B * L, 128), jnp.float32),
        scratch_types=[
            pltpu.VMEM((_ROWS_PER_W,), jnp.int32),
            pltpu.VMEM((_ROWS_PER_W, 128), jnp.float32),
            pltpu.SemaphoreType.DMA,
        ],
    )
    def gather_kernel(table_hbm, idx_hbm, out_hbm, idx_v, rows_v, sem):
        wid = lax.axis_index("s") * _SC_CORES + lax.axis_index("c")
        base = wid * _ROWS_PER_W
        pltpu.sync_copy(idx_hbm.at[pl.ds(base, _ROWS_PER_W)], idx_v)
        pltpu.async_copy(table_hbm.at[idx_v], rows_v, sem).wait()
        pltpu.sync_copy(rows_v, out_hbm.at[pl.ds(base, _ROWS_PER_W)])

    return gather_kernel


def _word_gather(table, ids):
    return _make_word_gather()(table, ids)


_G = 8


def _tc_body(h0_ref, adjt_ref, ett_ref, pe_ref, eemb_ref, wq_ref,
             wk_ref, wv_ref, wo_ref, lng_ref, lnb_ref, wout_ref, bout_ref,
             out_ref):
    G = range(_G)
    scale = jnp.float32(1.0 / np.sqrt(DH))
    hs = [h0_ref[g][:, :D] + pe_ref[...] for g in G]
    edge_ok = [adjt_ref[g] > 0 for g in G]
    bits = [[None] * 5 for g in G]
    for g in G:
        r = ett_ref[g].astype(jnp.bfloat16)
        for j in (4, 3, 2, 1, 0):
            bits[g][j] = r >= jnp.bfloat16(1 << j)
            if j:
                r = r - jnp.where(bits[g][j], jnp.bfloat16(1 << j),
                                  jnp.bfloat16(0))

    for i in range(NL):
        Wq = wq_ref[i]
        Wk = wk_ref[i]
        Wv = wv_ref[i]
        eproj = jnp.dot(eemb_ref[...], Wk,
                        preferred_element_type=jnp.float32)
        qs = [jnp.dot(hs[g], Wq, preferred_element_type=jnp.float32)
              for g in G]
        ks = [jnp.dot(hs[g], Wk, preferred_element_type=jnp.float32)
              for g in G]
        vs = [jnp.dot(hs[g], Wv, preferred_element_type=jnp.float32)
              for g in G]
        sts = [[None] * NH for g in G]
        qets = [[None] * NH for g in G]
        for hh in range(NH):
            sl = slice(hh * DH, (hh + 1) * DH)
            for g in G:
                qets[g][hh] = lax.dot_general(
                    eproj[:, sl], qs[g][:, sl], (((1,), (1,)), ((), ())),
                    preferred_element_type=jnp.float32)
                sts[g][hh] = lax.dot_general(
                    ks[g][:, sl], qs[g][:, sl], (((1,), (1,)), ((), ())),
                    preferred_element_type=jnp.float32)
        for hh in range(NH):
            for g in G:
                qet16 = qets[g][hh].astype(jnp.bfloat16)
                cur = [qet16[t:t + 1, :] for t in range(NE)]
                for j in range(5):
                    cur = [jnp.where(bits[g][j], cur[2 * s + 1], cur[2 * s])
                           for s in range(len(cur) // 2)]
                sts[g][hh] = sts[g][hh] + cur[0].astype(jnp.float32)
        outs = [[None] * NH for g in G]
        for hh in range(NH):
            for g in G:
                st = jnp.where(edge_ok[g], sts[g][hh] * scale,
                               jnp.float32(-1e9))
                p = jnp.exp(st)
                p = p * (1.0 / jnp.sum(p, axis=0, keepdims=True))
                outs[g][hh] = lax.dot_general(
                    p, vs[g][:, hh * DH:(hh + 1) * DH],
                    (((0,), (0,)), ((), ())),
                    preferred_element_type=jnp.float32)
        for g in G:
            attn_out = jnp.concatenate(outs[g], axis=1)
            hs[g] = hs[g] + jnp.dot(attn_out, wo_ref[i],
                                    preferred_element_type=jnp.float32)
        for g in G:
            m = jnp.mean(hs[g], axis=1, keepdims=True)
            hc = hs[g] - m
            var = jnp.mean(hc * hc, axis=1, keepdims=True)
            hs[g] = hc * lax.rsqrt(var + 1e-5) * lng_ref[i] + lnb_ref[i]

    for g in G:
        cls = hs[g][0:1, :]
        logits = jnp.dot(cls, wout_ref[...],
                         preferred_element_type=jnp.float32) + bout_ref[...]
        out_ref[g] = jnp.pad(logits, ((0, 7), (0, 128 - NC)))


def _tc_forward(h0, adj, et, pe, edge_emb, Wq, Wk, Wv, Wo, ln_g, ln_b,
                W_out, b_out):
    full = lambda shape: pl.BlockSpec(shape, lambda b: (0,) * len(shape))
    padded = pl.pallas_call(
        _tc_body,
        grid=(B // _G,),
        in_specs=[
            pl.BlockSpec((_G, L, 128), lambda b: (b, 0, 0)),
            pl.BlockSpec((_G, L, L), lambda b: (b, 0, 0)),
            pl.BlockSpec((_G, L, L), lambda b: (b, 0, 0)),
            full((L, D)),
            full((NE, D)),
            full((NL, D, D)),
            full((NL, D, D)),
            full((NL, D, D)),
            full((NL, D, D)),
            full((NL, D)),
            full((NL, D)),
            full((D, NC)),
            full((1, NC)),
        ],
        out_specs=pl.BlockSpec((_G, 8, 128), lambda b: (b, 0, 0)),
        out_shape=jax.ShapeDtypeStruct((B, 8, 128), jnp.float32),
    )(h0, adj, et, pe, edge_emb, Wq, Wk, Wv, Wo, ln_g, ln_b, W_out, b_out)
    return padded[:, 0, :NC]


def kernel(word_ids, adj, edge_types, word_emb, edge_emb, Wq, Wk, Wv, Wo,
           ln_g, ln_b, W_out, b_out):
    ids = word_ids.reshape(B * L).astype(jnp.int32)
    table = jnp.pad(word_emb, ((0, 0), (0, 128 - D)))
    h0 = _word_gather(table, ids).reshape(B, L, 128)
    pe = jnp.asarray(_PE)
    adjt = jnp.swapaxes(adj, 1, 2)
    ett = jnp.swapaxes(edge_types, 1, 2)
    return _tc_forward(h0, adjt, ett, pe, edge_emb, Wq, Wk, Wv, Wo,
                       ln_g, ln_b, W_out, b_out.reshape(1, NC))
